# baseline (speedup 1.0000x reference)
import functools

import jax
import jax.numpy as jnp
from jax import lax
from jax.experimental import pallas as pl
from jax.experimental.pallas import tpu as pltpu
from jax.experimental.pallas import tpu_sc as plsc

N = 10000
E = 160000
EP = 163840
NC = 2
NS = 16
MINOR = 128
NA = N + 8
ZROWS = 200
NZ = N // ZROWS
ZSUB = 20

_mesh = functools.partial(
    plsc.VectorSubcoreMesh,
    core_axis_name="c", subcore_axis_name="s", num_cores=NC, num_subcores=NS)


def _zero_rows(buf, nrows, dc):
  def body(i, _):
    for j in range(dc // 16):
      buf[i, pl.ds(j * 16, 16)] = jnp.zeros((16,), jnp.float32)
    return 0
  lax.fori_loop(0, nrows, body, 0, unroll=False)


@functools.cache
def _make_deg_kernel():
  epl = EP // (NC * NS)
  nb = epl // MINOR
  dc = 128

  @functools.partial(
      pl.kernel,
      out_type=jax.ShapeDtypeStruct((NC, N, dc), jnp.float32),
      mesh=_mesh(),
      scratch_types=[
          pltpu.VMEM((nb, MINOR), jnp.int32),
          pltpu.VMEM((MINOR, dc), jnp.float32),
          pltpu.VMEM((ZSUB, dc), jnp.float32),
          pltpu.VMEM_SHARED((NA, dc), jnp.float32),
      ],
  )
  def deg_kernel(dst_hbm, out_hbm, idx_d, obuf, zbuf, acc):
    cid = lax.axis_index("c")
    sid = lax.axis_index("s")
    wid = cid * NS + sid

    _zero_rows(zbuf, ZSUB, dc)
    one_hot = jnp.where(lax.iota(jnp.int32, 16) == 0,
                        jnp.float32(1.0), jnp.float32(0.0))
    def fill(i, _):
      obuf[i, pl.ds(0, 16)] = one_hot
      for j in range(1, dc // 16):
        obuf[i, pl.ds(j * 16, 16)] = jnp.zeros((16,), jnp.float32)
      return 0
    lax.fori_loop(0, MINOR, fill, 0, unroll=False)

    for j in range((NZ + NS - 1) // NS):
      rc = j * NS + sid
      @pl.when(rc < NZ)
      def _():
        for k in range(ZROWS // ZSUB):
          pltpu.sync_copy(zbuf, acc.at[pl.ds(rc * ZROWS + k * ZSUB, ZSUB)])
    plsc.subcore_barrier()

    pltpu.sync_copy(dst_hbm.at[wid], idx_d)
    def body(b, _):
      pltpu.sync_copy(obuf, acc.at[idx_d.at[b]], add=True)
      return 0
    lax.fori_loop(0, nb, body, 0, unroll=False)
    plsc.subcore_barrier()

    for j in range((NZ + NS - 1) // NS):
      rc = j * NS + sid
      @pl.when(rc < NZ)
      def _():
        pltpu.sync_copy(acc.at[pl.ds(rc * ZROWS, ZROWS)],
                        out_hbm.at[cid].at[pl.ds(rc * ZROWS, ZROWS)])

  return deg_kernel


@functools.cache
def _make_agg_kernel(nchunks, dc):
  split_edges = (nchunks == 1)
  if split_edges:
    epl = EP // (NC * NS)
    chunks_per_core = 1
  else:
    assert nchunks % NC == 0
    epl = EP // NS
    chunks_per_core = nchunks // NC
  nb = epl // MINOR
  nout = NC if split_edges else nchunks

  @functools.partial(
      pl.kernel,
      out_type=jax.ShapeDtypeStruct((nout, N, dc), jnp.float32),
      mesh=_mesh(),
      scratch_types=[
          pltpu.VMEM((nb, MINOR), jnp.int32),
          pltpu.VMEM((nb, MINOR), jnp.int32),
          pltpu.VMEM((MINOR, dc), jnp.float32),
          pltpu.VMEM((ZSUB, dc), jnp.float32),
          pltpu.VMEM_SHARED((NA, dc), jnp.float32),
      ],
  )
  def agg_kernel(y_hbm, src_hbm, dst_hbm, out_hbm,
                 idx_s, idx_d, gbuf, zbuf, acc):
    cid = lax.axis_index("c")
    sid = lax.axis_index("s")

    _zero_rows(zbuf, ZSUB, dc)
    if split_edges:
      pltpu.sync_copy(dst_hbm.at[cid * NS + sid], idx_d)
    else:
      pltpu.sync_copy(dst_hbm.at[sid], idx_d)

    for i in range(chunks_per_core):
      chunk = i * NC + cid
      for j in range((NZ + NS - 1) // NS):
        rc = j * NS + sid
        @pl.when(rc < NZ)
        def _():
          for k in range(ZROWS // ZSUB):
            pltpu.sync_copy(zbuf, acc.at[pl.ds(rc * ZROWS + k * ZSUB, ZSUB)])
      plsc.subcore_barrier()

      if split_edges:
        pltpu.sync_copy(src_hbm.at[0].at[cid * NS + sid], idx_s)
      else:
        pltpu.sync_copy(src_hbm.at[chunk].at[sid], idx_s)

      def body(b, _):
        pltpu.sync_copy(y_hbm.at[idx_s.at[b]], gbuf)
        pltpu.sync_copy(gbuf, acc.at[idx_d.at[b]], add=True)
        return 0
      lax.fori_loop(0, nb, body, 0, unroll=False)
      plsc.subcore_barrier()

      slot = cid if split_edges else chunk
      for j in range((NZ + NS - 1) // NS):
        rc = j * NS + sid
        @pl.when(rc < NZ)
        def _():
          pltpu.sync_copy(acc.at[pl.ds(rc * ZROWS, ZROWS)],
                          out_hbm.at[slot].at[pl.ds(rc * ZROWS, ZROWS)])
      plsc.subcore_barrier()

  return agg_kernel


BR = 1000


def _tc1_body(hist_ref, x_ref, dinv_ref, x0_ref):
  deg = (jnp.float32(1.0) + hist_ref[0, :, 0:1] + hist_ref[1, :, 0:1])
  dinv = lax.rsqrt(deg)
  dinv_ref[:] = dinv
  for c in range(2):
    x0_ref[c] = x_ref[:, c * 128:(c + 1) * 128] * dinv


def _tc1(hist, x):
  return pl.pallas_call(
      _tc1_body,
      grid=(N // BR,),
      in_specs=[
          pl.BlockSpec((NC, BR, 128), lambda r: (0, r, 0)),
          pl.BlockSpec((BR, 256), lambda r: (r, 0)),
      ],
      out_specs=[
          pl.BlockSpec((BR, 1), lambda r: (r, 0)),
          pl.BlockSpec((2, BR, 128), lambda r: (0, r, 0)),
      ],
      out_shape=[
          jax.ShapeDtypeStruct((N, 1), jnp.float32),
          jax.ShapeDtypeStruct((2, N, 128), jnp.float32),
      ],
  )(hist, x)


def _make_tc_mid(nc_in, nc_out, k_in, k_out):
  def body(s_ref, x_ref, dinv_ref, w_ref, b_ref, out_ref):
    d = dinv_ref[:]
    acc = jnp.zeros((BR, k_out), jnp.float32)
    for c in range(nc_in):
      u = (s_ref[c] + x_ref[c]) * d
      acc = acc + jnp.dot(u, w_ref[c * 128:(c + 1) * 128, :],
                          preferred_element_type=jnp.float32)
    h = jnp.maximum(acc + b_ref[:], 0.0) * d
    for c in range(nc_out):
      out_ref[c] = h[:, c * 128:(c + 1) * 128]

  def call(s, x, dinv, w, b):
    return pl.pallas_call(
        body,
        grid=(N // BR,),
        in_specs=[
            pl.BlockSpec((nc_in, BR, 128), lambda r: (0, r, 0)),
            pl.BlockSpec((nc_in, BR, 128), lambda r: (0, r, 0)),
            pl.BlockSpec((BR, 1), lambda r: (r, 0)),
            pl.BlockSpec((k_in, k_out), lambda r: (0, 0)),
            pl.BlockSpec((1, k_out), lambda r: (0, 0)),
        ],
        out_specs=pl.BlockSpec((nc_out, BR, 128), lambda r: (0, r, 0)),
        out_shape=jax.ShapeDtypeStruct((nc_out, N, 128), jnp.float32),
    )(s, x, dinv, w, b)

  return call


def _tc2(s, x, dinv, w, b):
  return _make_tc_mid(2, 8, 256, 1024)(s, x, dinv, w, b)


def _tc3_body(s_ref, x_ref, dinv_ref, w2_ref, b2_ref, wcat_ref, x2_ref):
  d = dinv_ref[:]
  acc = jnp.zeros((BR, 1024), jnp.float32)
  for c in range(8):
    u = (s_ref[c] + x_ref[c]) * d
    acc = acc + jnp.dot(u, w2_ref[c * 128:(c + 1) * 128, :],
                        preferred_element_type=jnp.float32)
  h2 = jnp.maximum(acc + b2_ref[:], 0.0)
  z = jnp.dot(h2, wcat_ref[:], preferred_element_type=jnp.float32)
  x2_ref[:] = z * d


def _tc3(s, x, dinv, w2, b2, wcat):
  return pl.pallas_call(
      _tc3_body,
      grid=(N // BR,),
      in_specs=[
          pl.BlockSpec((8, BR, 128), lambda r: (0, r, 0)),
          pl.BlockSpec((8, BR, 128), lambda r: (0, r, 0)),
          pl.BlockSpec((BR, 1), lambda r: (r, 0)),
          pl.BlockSpec((1024, 1024), lambda r: (0, 0)),
          pl.BlockSpec((1, 1024), lambda r: (0, 0)),
          pl.BlockSpec((1024, 128), lambda r: (0, 0)),
      ],
      out_specs=pl.BlockSpec((BR, 128), lambda r: (r, 0)),
      out_shape=jax.ShapeDtypeStruct((N, 128), jnp.float32),
  )(s, x, dinv, w2, b2, wcat)


def _tc4_body(s_ref, x_ref, dinv_ref, bcat_ref, out_ref):
  out_ref[:] = ((s_ref[0] + s_ref[1] + x_ref[:]) * dinv_ref[:]) + bcat_ref[:]


def _tc4(s, x, dinv, bcat):
  return pl.pallas_call(
      _tc4_body,
      grid=(N // BR,),
      in_specs=[
          pl.BlockSpec((NC, BR, 128), lambda r: (0, r, 0)),
          pl.BlockSpec((BR, 128), lambda r: (r, 0)),
          pl.BlockSpec((BR, 1), lambda r: (r, 0)),
          pl.BlockSpec((1, 128), lambda r: (0, 0)),
      ],
      out_specs=pl.BlockSpec((BR, 128), lambda r: (r, 0)),
      out_shape=jax.ShapeDtypeStruct((N, 128), jnp.float32),
  )(s, x, dinv, bcat)


@jax.jit
def kernel(x, edge_index, W1, b1, W2, b2, Wmu, bmu, Wls, bls):
  src = jnp.concatenate(
      [edge_index[0], jnp.zeros((EP - E,), jnp.int32)])
  dst = jnp.concatenate(
      [edge_index[1], jnp.full((EP - E,), N, jnp.int32)])

  nbA = (EP // NS) // MINOR
  nbB = (EP // (NC * NS)) // MINOR
  dstA = dst.reshape(NS, nbA, MINOR)
  dstB = dst.reshape(NC * NS, nbB, MINOR)
  srcB = src.reshape(1, NC * NS, nbB, MINOR)
  off2 = src[None, :] + (jnp.arange(2, dtype=jnp.int32) * N)[:, None]
  srcA2 = off2.reshape(2, NS, nbA, MINOR)
  off8 = src[None, :] + (jnp.arange(8, dtype=jnp.int32) * N)[:, None]
  srcA8 = off8.reshape(8, NS, nbA, MINOR)

  wcat = jnp.concatenate(
      [Wmu, Wls, jnp.zeros((Wmu.shape[0], 120), jnp.float32)], axis=1)
  bcat = jnp.concatenate(
      [bmu, bls, jnp.zeros((120,), jnp.float32)]).reshape(1, 128)

  hist = _make_deg_kernel()(dstB)
  dinv, x0 = _tc1(hist, x)

  s0 = _make_agg_kernel(2, 128)(x0.reshape(2 * N, 128), srcA2, dstA)
  x1 = _tc2(s0, x0, dinv, W1, b1.reshape(1, 1024))

  s1 = _make_agg_kernel(8, 128)(x1.reshape(8 * N, 128), srcA8, dstA)
  x2 = _tc3(s1, x1, dinv, W2, b2.reshape(1, 1024), wcat)

  s2 = _make_agg_kernel(1, 128)(x2, srcB, dstB)
  out = _tc4(s2, x2, dinv, bcat)

  return out[:, :4], out[:, 4:8]


# speedup vs baseline: 1.1889x; 1.1889x over previous
import functools

import jax
import jax.numpy as jnp
from jax import lax
from jax.experimental import pallas as pl
from jax.experimental.pallas import tpu as pltpu
from jax.experimental.pallas import tpu_sc as plsc

N = 10000
E = 160000
EP = 163840
NC = 2
NS = 16
MINOR = 128
NA = N + 8
ZROWS = 200
NZ = N // ZROWS
ZSUB = 20

_mesh = functools.partial(
    plsc.VectorSubcoreMesh,
    core_axis_name="c", subcore_axis_name="s", num_cores=NC, num_subcores=NS)


def _zero_rows(buf, nrows, dc):
  def body(i, _):
    for j in range(dc // 16):
      buf[i, pl.ds(j * 16, 16)] = jnp.zeros((16,), jnp.float32)
    return 0
  lax.fori_loop(0, nrows, body, 0, unroll=False)


@functools.cache
def _make_deg_kernel():
  epl = EP // (NC * NS)
  nb = epl // MINOR
  dc = 128

  @functools.partial(
      pl.kernel,
      out_type=jax.ShapeDtypeStruct((NC, N, dc), jnp.float32),
      mesh=_mesh(),
      scratch_types=[
          pltpu.VMEM((nb, MINOR), jnp.int32),
          pltpu.VMEM((MINOR, dc), jnp.float32),
          pltpu.VMEM((ZSUB, dc), jnp.float32),
          pltpu.VMEM_SHARED((NA, dc), jnp.float32),
      ],
  )
  def deg_kernel(dst_hbm, out_hbm, idx_d, obuf, zbuf, acc):
    cid = lax.axis_index("c")
    sid = lax.axis_index("s")
    wid = cid * NS + sid

    _zero_rows(zbuf, ZSUB, dc)
    one_hot = jnp.where(lax.iota(jnp.int32, 16) == 0,
                        jnp.float32(1.0), jnp.float32(0.0))
    def fill(i, _):
      obuf[i, pl.ds(0, 16)] = one_hot
      for j in range(1, dc // 16):
        obuf[i, pl.ds(j * 16, 16)] = jnp.zeros((16,), jnp.float32)
      return 0
    lax.fori_loop(0, MINOR, fill, 0, unroll=False)

    for j in range((NZ + NS - 1) // NS):
      rc = j * NS + sid
      @pl.when(rc < NZ)
      def _():
        for k in range(ZROWS // ZSUB):
          pltpu.sync_copy(zbuf, acc.at[pl.ds(rc * ZROWS + k * ZSUB, ZSUB)])
    plsc.subcore_barrier()

    pltpu.sync_copy(dst_hbm.at[wid], idx_d)
    def body(b, _):
      pltpu.sync_copy(obuf, acc.at[idx_d.at[b]], add=True)
      return 0
    lax.fori_loop(0, nb, body, 0, unroll=False)
    plsc.subcore_barrier()

    for j in range((NZ + NS - 1) // NS):
      rc = j * NS + sid
      @pl.when(rc < NZ)
      def _():
        pltpu.sync_copy(acc.at[pl.ds(rc * ZROWS, ZROWS)],
                        out_hbm.at[cid].at[pl.ds(rc * ZROWS, ZROWS)])

  return deg_kernel


@functools.cache
def _make_agg_kernel(nchunks, dc):
  split_edges = (nchunks == 1)
  if split_edges:
    epl = EP // (NC * NS)
    chunks_per_core = 1
  else:
    assert nchunks % NC == 0
    epl = EP // NS
    chunks_per_core = nchunks // NC
  nb = epl // MINOR
  nhalves = 1 if split_edges else 2
  nh = nb // nhalves
  nout = NC if split_edges else nchunks

  @functools.partial(
      pl.kernel,
      out_type=jax.ShapeDtypeStruct((nout, N, dc), jnp.float32),
      mesh=_mesh(),
      scratch_types=[
          pltpu.VMEM((nh, MINOR), jnp.int32),
          pltpu.VMEM((nh, MINOR), jnp.int32),
          pltpu.VMEM((MINOR, dc), jnp.float32),
          pltpu.VMEM((MINOR, dc), jnp.float32),
          pltpu.VMEM((ZSUB, dc), jnp.float32),
          pltpu.VMEM_SHARED((NA, dc), jnp.float32),
          pltpu.SemaphoreType.DMA,
          pltpu.SemaphoreType.DMA,
      ],
  )
  def agg_kernel(y_hbm, src_hbm, dst_hbm, out_hbm,
                 idx_s, idx_d, g0, g1, zbuf, acc, sem0, sem1):
    cid = lax.axis_index("c")
    sid = lax.axis_index("s")

    _zero_rows(zbuf, ZSUB, dc)

    for i in range(chunks_per_core):
      chunk = i * NC + cid
      for j in range((NZ + NS - 1) // NS):
        rc = j * NS + sid
        @pl.when(rc < NZ)
        def _():
          for k in range(ZROWS // ZSUB):
            pltpu.sync_copy(zbuf, acc.at[pl.ds(rc * ZROWS + k * ZSUB, ZSUB)])
      plsc.subcore_barrier()

      for half in range(nhalves):
        hs = pl.ds(half * nh, nh)
        if split_edges:
          pltpu.sync_copy(src_hbm.at[0].at[cid * NS + sid].at[hs], idx_s)
          pltpu.sync_copy(dst_hbm.at[cid * NS + sid].at[hs], idx_d)
        else:
          pltpu.sync_copy(src_hbm.at[chunk].at[sid].at[hs], idx_s)
          pltpu.sync_copy(dst_hbm.at[sid].at[hs], idx_d)

        pltpu.async_copy(y_hbm.at[idx_s.at[0]], g0, sem0)
        pltpu.async_copy(y_hbm.at[idx_s.at[1]], g1, sem1)
        def body(h, _):
          b = h * 2
          for j, (g, sem) in enumerate(((g0, sem0), (g1, sem1))):
            pltpu.make_async_copy(y_hbm.at[idx_s.at[b + j]], g, sem).wait()
            pltpu.sync_copy(g, acc.at[idx_d.at[b + j]], add=True)
            @pl.when(b + j + 2 < nh)
            def _():
              pltpu.async_copy(y_hbm.at[idx_s.at[b + j + 2]], g, sem)
          return 0
        lax.fori_loop(0, nh // 2, body, 0, unroll=False)
      plsc.subcore_barrier()

      slot = cid if split_edges else chunk
      for j in range((NZ + NS - 1) // NS):
        rc = j * NS + sid
        @pl.when(rc < NZ)
        def _():
          pltpu.sync_copy(acc.at[pl.ds(rc * ZROWS, ZROWS)],
                          out_hbm.at[slot].at[pl.ds(rc * ZROWS, ZROWS)])
      plsc.subcore_barrier()

  return agg_kernel


BR = 1000


def _tc1_body(hist_ref, x_ref, dinv_ref, x0_ref):
  deg = (jnp.float32(1.0) + hist_ref[0, :, 0:1] + hist_ref[1, :, 0:1])
  dinv = lax.rsqrt(deg)
  dinv_ref[:] = dinv
  for c in range(2):
    x0_ref[c] = x_ref[:, c * 128:(c + 1) * 128] * dinv


def _tc1(hist, x):
  return pl.pallas_call(
      _tc1_body,
      grid=(N // BR,),
      in_specs=[
          pl.BlockSpec((NC, BR, 128), lambda r: (0, r, 0)),
          pl.BlockSpec((BR, 256), lambda r: (r, 0)),
      ],
      out_specs=[
          pl.BlockSpec((BR, 1), lambda r: (r, 0)),
          pl.BlockSpec((2, BR, 128), lambda r: (0, r, 0)),
      ],
      out_shape=[
          jax.ShapeDtypeStruct((N, 1), jnp.float32),
          jax.ShapeDtypeStruct((2, N, 128), jnp.float32),
      ],
  )(hist, x)


def _make_tc_mid(nc_in, nc_out, k_in, k_out):
  def body(s_ref, x_ref, dinv_ref, w_ref, b_ref, out_ref):
    d = dinv_ref[:]
    acc = jnp.zeros((BR, k_out), jnp.float32)
    for c in range(nc_in):
      u = (s_ref[c] + x_ref[c]) * d
      acc = acc + jnp.dot(u, w_ref[c * 128:(c + 1) * 128, :],
                          preferred_element_type=jnp.float32)
    h = jnp.maximum(acc + b_ref[:], 0.0) * d
    for c in range(nc_out):
      out_ref[c] = h[:, c * 128:(c + 1) * 128]

  def call(s, x, dinv, w, b):
    return pl.pallas_call(
        body,
        grid=(N // BR,),
        in_specs=[
            pl.BlockSpec((nc_in, BR, 128), lambda r: (0, r, 0)),
            pl.BlockSpec((nc_in, BR, 128), lambda r: (0, r, 0)),
            pl.BlockSpec((BR, 1), lambda r: (r, 0)),
            pl.BlockSpec((k_in, k_out), lambda r: (0, 0)),
            pl.BlockSpec((1, k_out), lambda r: (0, 0)),
        ],
        out_specs=pl.BlockSpec((nc_out, BR, 128), lambda r: (0, r, 0)),
        out_shape=jax.ShapeDtypeStruct((nc_out, N, 128), jnp.float32),
    )(s, x, dinv, w, b)

  return call


def _tc2(s, x, dinv, w, b):
  return _make_tc_mid(2, 8, 256, 1024)(s, x, dinv, w, b)


def _tc3_body(s_ref, x_ref, dinv_ref, w2_ref, b2_ref, wcat_ref, x2_ref):
  d = dinv_ref[:]
  acc = jnp.zeros((BR, 1024), jnp.float32)
  for c in range(8):
    u = (s_ref[c] + x_ref[c]) * d
    acc = acc + jnp.dot(u, w2_ref[c * 128:(c + 1) * 128, :],
                        preferred_element_type=jnp.float32)
  h2 = jnp.maximum(acc + b2_ref[:], 0.0)
  z = jnp.dot(h2, wcat_ref[:], preferred_element_type=jnp.float32)
  x2_ref[:] = z * d


def _tc3(s, x, dinv, w2, b2, wcat):
  return pl.pallas_call(
      _tc3_body,
      grid=(N // BR,),
      in_specs=[
          pl.BlockSpec((8, BR, 128), lambda r: (0, r, 0)),
          pl.BlockSpec((8, BR, 128), lambda r: (0, r, 0)),
          pl.BlockSpec((BR, 1), lambda r: (r, 0)),
          pl.BlockSpec((1024, 1024), lambda r: (0, 0)),
          pl.BlockSpec((1, 1024), lambda r: (0, 0)),
          pl.BlockSpec((1024, 128), lambda r: (0, 0)),
      ],
      out_specs=pl.BlockSpec((BR, 128), lambda r: (r, 0)),
      out_shape=jax.ShapeDtypeStruct((N, 128), jnp.float32),
  )(s, x, dinv, w2, b2, wcat)


def _tc4_body(s_ref, x_ref, dinv_ref, bcat_ref, out_ref):
  out_ref[:] = ((s_ref[0] + s_ref[1] + x_ref[:]) * dinv_ref[:]) + bcat_ref[:]


def _tc4(s, x, dinv, bcat):
  return pl.pallas_call(
      _tc4_body,
      grid=(N // BR,),
      in_specs=[
          pl.BlockSpec((NC, BR, 128), lambda r: (0, r, 0)),
          pl.BlockSpec((BR, 128), lambda r: (r, 0)),
          pl.BlockSpec((BR, 1), lambda r: (r, 0)),
          pl.BlockSpec((1, 128), lambda r: (0, 0)),
      ],
      out_specs=pl.BlockSpec((BR, 128), lambda r: (r, 0)),
      out_shape=jax.ShapeDtypeStruct((N, 128), jnp.float32),
  )(s, x, dinv, bcat)


@jax.jit
def kernel(x, edge_index, W1, b1, W2, b2, Wmu, bmu, Wls, bls):
  src = jnp.concatenate(
      [edge_index[0], jnp.zeros((EP - E,), jnp.int32)])
  dst = jnp.concatenate(
      [edge_index[1], jnp.full((EP - E,), N, jnp.int32)])

  nbA = (EP // NS) // MINOR
  nbB = (EP // (NC * NS)) // MINOR
  dstA = dst.reshape(NS, nbA, MINOR)
  dstB = dst.reshape(NC * NS, nbB, MINOR)
  srcB = src.reshape(1, NC * NS, nbB, MINOR)
  off2 = src[None, :] + (jnp.arange(2, dtype=jnp.int32) * N)[:, None]
  srcA2 = off2.reshape(2, NS, nbA, MINOR)
  off8 = src[None, :] + (jnp.arange(8, dtype=jnp.int32) * N)[:, None]
  srcA8 = off8.reshape(8, NS, nbA, MINOR)

  wcat = jnp.concatenate(
      [Wmu, Wls, jnp.zeros((Wmu.shape[0], 120), jnp.float32)], axis=1)
  bcat = jnp.concatenate(
      [bmu, bls, jnp.zeros((120,), jnp.float32)]).reshape(1, 128)

  hist = _make_deg_kernel()(dstB)
  dinv, x0 = _tc1(hist, x)

  s0 = _make_agg_kernel(2, 128)(x0.reshape(2 * N, 128), srcA2, dstA)
  x1 = _tc2(s0, x0, dinv, W1, b1.reshape(1, 1024))

  s1 = _make_agg_kernel(8, 128)(x1.reshape(8 * N, 128), srcA8, dstA)
  x2 = _tc3(s1, x1, dinv, W2, b2.reshape(1, 1024), wcat)

  s2 = _make_agg_kernel(1, 128)(x2, srcB, dstB)
  out = _tc4(s2, x2, dinv, bcat)

  return out[:, :4], out[:, 4:8]
